# baseline (speedup 1.0000x reference)
import functools

import jax
import jax.numpy as jnp
from jax import lax
from jax.experimental import pallas as pl
from jax.experimental.pallas import tpu as pltpu
from jax.experimental.pallas import tpu_sc as plsc

N_NODES = 10000
N_EDGES = 320000
D_FEAT = 128

_NC = 2
_NS = 16
_NW = _NC * _NS

_E_PER_W = N_EDGES // _NW
_CHUNK = 80
_K_CH = _E_PER_W // _CHUNK

_G_ROWS = N_EDGES // 4

_NODE_BLK = 1000
_EDGE_BLK = 2000


def _node_body(x_ref, w1_ref, b1_ref, w2_ref, b2_ref, wba_ref, wbb_ref,
               val_ref, pv_ref, xa_ref, xb_ref):
    x = x_ref[...]
    h = jnp.maximum(jnp.dot(x, w1_ref[...],
                            preferred_element_type=jnp.float32) + b1_ref[...],
                    0.0)
    logits = jnp.dot(h, w2_ref[...],
                     preferred_element_type=jnp.float32) + b2_ref[...]
    m = jnp.max(logits, axis=-1, keepdims=True)
    e = jnp.exp(logits - m)
    val_ref[...] = e / jnp.sum(e, axis=-1, keepdims=True)
    idx8 = lax.broadcasted_iota(jnp.int32, logits.shape, 1)
    big = jnp.where(logits == m, idx8, logits.shape[-1])
    am = jnp.min(big, axis=-1, keepdims=True)
    pv_ref[...] = am.astype(jnp.float32) + 1.0
    xa_ref[...] = jnp.dot(x, wba_ref[...], preferred_element_type=jnp.float32)
    xb_ref[...] = jnp.dot(x, wbb_ref[...], preferred_element_type=jnp.float32)


def _node_stage(x, W1, b1, W2, b2, Wba, Wbb):
    nblk = N_NODES // _NODE_BLK
    full = lambda i: (0, 0)
    return pl.pallas_call(
        _node_body,
        grid=(nblk,),
        in_specs=[
            pl.BlockSpec((_NODE_BLK, D_FEAT), lambda i: (i, 0)),
            pl.BlockSpec((D_FEAT, 32), full),
            pl.BlockSpec((1, 32), full),
            pl.BlockSpec((32, 8), full),
            pl.BlockSpec((1, 8), full),
            pl.BlockSpec((D_FEAT, 32), full),
            pl.BlockSpec((D_FEAT, 32), full),
        ],
        out_specs=[
            pl.BlockSpec((_NODE_BLK, 8), lambda i: (i, 0)),
            pl.BlockSpec((_NODE_BLK, 1), lambda i: (i, 0)),
            pl.BlockSpec((_NODE_BLK, 32), lambda i: (i, 0)),
            pl.BlockSpec((_NODE_BLK, 32), lambda i: (i, 0)),
        ],
        out_shape=[
            jax.ShapeDtypeStruct((N_NODES, 8), jnp.float32),
            jax.ShapeDtypeStruct((N_NODES, 1), jnp.float32),
            jax.ShapeDtypeStruct((N_NODES, 32), jnp.float32),
            jax.ShapeDtypeStruct((N_NODES, 32), jnp.float32),
        ],
    )(x, W1, b1, W2, b2, Wba, Wbb)


@functools.lru_cache(maxsize=None)
def _sc_mesh():
    return plsc.VectorSubcoreMesh(core_axis_name="c", subcore_axis_name="s")


@functools.lru_cache(maxsize=None)
def _build_sc_gather():
    return pl.kernel(
        _sc_gather_body,
        out_type=jax.ShapeDtypeStruct((2 * N_EDGES, 32), jnp.float32),
        mesh=_sc_mesh(),
        scratch_types=[
            pltpu.VMEM((_E_PER_W,), jnp.int32),
            pltpu.VMEM((2, _CHUNK, 32), jnp.float32),
            pltpu.SemaphoreType.DMA,
            pltpu.SemaphoreType.DMA,
        ],
        compiler_params=pltpu.CompilerParams(use_tc_tiling_on_sc=False),
    )


def _sc_gather_body(xa_hbm, xb_hbm, ei_hbm, out_hbm, idx_v, rows_v, gsem,
                    ssem):
    wid = lax.axis_index("s") * _NC + lax.axis_index("c")
    base = wid * _E_PER_W

    def run_side(side, table, out_base):
        pltpu.sync_copy(ei_hbm.at[side, pl.ds(base, _E_PER_W)], idx_v)

        def idx_at(j):
            return idx_v.at[pl.ds(pl.multiple_of(j * _CHUNK, 8), _CHUNK)]

        def out_at(j):
            return out_hbm.at[
                pl.ds(pl.multiple_of(out_base + j * _CHUNK, 8), _CHUNK)]

        pltpu.async_copy(table.at[idx_at(0)], rows_v.at[0], gsem)

        def body(j, _):
            slot = lax.rem(j, 2)

            @pl.when(j + 1 < _K_CH)
            def _():
                @pl.when(j >= 1)
                def _():
                    pltpu.make_async_copy(rows_v.at[lax.rem(j + 1, 2)],
                                          out_at(j - 1), ssem).wait()

                pltpu.async_copy(table.at[idx_at(j + 1)],
                                 rows_v.at[lax.rem(j + 1, 2)], gsem)

            pltpu.make_async_copy(table.at[idx_at(j)], rows_v.at[slot],
                                  gsem).wait()
            pltpu.async_copy(rows_v.at[slot], out_at(j), ssem)
            return 0

        lax.fori_loop(0, _K_CH, body, 0)
        pltpu.make_async_copy(rows_v.at[0], out_at(_K_CH - 1), ssem).wait()
        pltpu.make_async_copy(rows_v.at[0], out_at(_K_CH - 1), ssem).wait()

    run_side(0, xa_hbm, base)
    run_side(1, xb_hbm, N_EDGES + base)


def _edge_body(ga_ref, gb_ref, bb1_ref, wb2_ref, bb2_ref, gsum_ref, bt_ref):
    hb = jnp.maximum(ga_ref[...] + gb_ref[...] + bb1_ref[...], 0.0)
    logits = jnp.dot(hb, wb2_ref[...],
                     preferred_element_type=jnp.float32) + bb2_ref[...]
    m = jnp.max(logits, axis=-1, keepdims=True)
    e = jnp.exp(logits - m)
    s = jnp.dot(e, gsum_ref[...], preferred_element_type=jnp.float32)
    bt = e / s
    bt_ref[...] = jnp.pad(bt, ((0, 0), (0, 112)))


def _edge_stage(gout, bb1_t4, Wb2bd, bb2_t4, gsum):
    nblk = _G_ROWS // _EDGE_BLK
    full = lambda i: (0, 0)
    return pl.pallas_call(
        _edge_body,
        grid=(nblk,),
        in_specs=[
            pl.BlockSpec((_EDGE_BLK, 128), lambda i: (i, 0)),
            pl.BlockSpec((_EDGE_BLK, 128), lambda i: (i + nblk, 0)),
            pl.BlockSpec((1, 128), full),
            pl.BlockSpec((128, 16), full),
            pl.BlockSpec((1, 16), full),
            pl.BlockSpec((16, 16), full),
        ],
        out_specs=[
            pl.BlockSpec((_EDGE_BLK, 128), lambda i: (i, 0)),
        ],
        out_shape=[
            jax.ShapeDtypeStruct((_G_ROWS, 128), jnp.float32),
        ],
    )(gout, gout, bb1_t4, Wb2bd, bb2_t4, gsum)


_R_PER_W = _G_ROWS // _NW


@functools.lru_cache(maxsize=None)
def _build_sc_scatter():
    return pl.kernel(
        _sc_scatter_body,
        out_type=[
            jax.ShapeDtypeStruct((_G_ROWS, 16), jnp.float32),
            jax.ShapeDtypeStruct((_NW * N_NODES,), jnp.float32),
        ],
        mesh=_sc_mesh(),
        scratch_types=[
            pltpu.VMEM((_E_PER_W,), jnp.int32),
            pltpu.VMEM((_R_PER_W, 16), jnp.float32),
            pltpu.VMEM((N_NODES,), jnp.float32),
        ],
        compiler_params=pltpu.CompilerParams(use_tc_tiling_on_sc=False,
                                             needs_layout_passes=False),
    )


def _sc_scatter_body(ei_hbm, btp_hbm, bt_hbm, deg_hbm, idx_v, bt_v, acc_v):
    wid = lax.axis_index("s") * _NC + lax.axis_index("c")
    rbase = wid * _R_PER_W
    ebase = wid * _E_PER_W
    pltpu.sync_copy(btp_hbm.at[pl.ds(rbase, _R_PER_W), pl.ds(0, 16)], bt_v)
    pltpu.sync_copy(bt_v, bt_hbm.at[pl.ds(rbase, _R_PER_W)])
    pltpu.sync_copy(ei_hbm.at[0, pl.ds(ebase, _E_PER_W)], idx_v)

    zero = jnp.zeros((16,), jnp.float32)

    def zbody(i, _):
        acc_v[pl.ds(pl.multiple_of(i * 16, 16), 16)] = zero
        return 0

    lax.fori_loop(0, N_NODES // 16, zbody, 0)

    lane = lax.iota(jnp.int32, 16)
    r4 = lane % 4
    wt16 = jnp.where(r4 == 3, 1.5, (r4 + 1).astype(jnp.float32))
    q4 = lane // 4

    def body(i, _):
        vals = bt_v[i] * wt16
        vidx = plsc.load_gather(idx_v, [i * 4 + q4])
        plsc.addupdate_scatter(acc_v, [vidx], vals)
        return 0

    lax.fori_loop(0, _R_PER_W, body, 0)
    pltpu.sync_copy(
        acc_v,
        deg_hbm.at[pl.ds(pl.multiple_of(wid * N_NODES, 16), N_NODES)])


def _final_body(part_ref, pv_ref, out_ref):
    deg = jnp.sum(part_ref[...], axis=0, keepdims=True)
    d = deg - pv_ref[...]
    out_ref[...] = jnp.sum(d * d, axis=-1, keepdims=True) / N_NODES


def _final_stage(partials, pv_row):
    return pl.pallas_call(
        _final_body,
        out_shape=jax.ShapeDtypeStruct((1, 1), jnp.float32),
    )(partials, pv_row)


def kernel(x, edge_index, W1, b1, W2, b2, Wb1, bb1, Wb2, bb2):
    ei = edge_index.astype(jnp.int32)

    valences, pv, xa, xb = _node_stage(
        x, W1, b1.reshape(1, 32), W2, b2.reshape(1, 8),
        Wb1[:D_FEAT], Wb1[D_FEAT:])

    gout = _build_sc_gather()(xa, xb, ei)
    gout = gout.reshape(2 * _G_ROWS, 128)

    eye4 = jnp.eye(4, dtype=jnp.float32)
    (btp,) = _edge_stage(
        gout,
        jnp.tile(bb1.reshape(1, 32), (1, 4)),
        jnp.kron(eye4, Wb2),
        jnp.tile(bb2.reshape(1, 4), (1, 4)),
        jnp.kron(eye4, jnp.ones((4, 4), jnp.float32)),
    )

    bt, partials = _build_sc_scatter()(ei, btp)
    partials = partials.reshape(_NW, N_NODES)

    vv = _final_stage(partials, pv.reshape(1, N_NODES))
    return (vv.reshape(()), valences, bt.reshape(N_EDGES, 4))
